# baseline (speedup 1.0000x reference)
import functools

import jax
import jax.numpy as jnp
from jax import lax
from jax.experimental import pallas as pl
from jax.experimental.pallas import tpu as pltpu
from jax.experimental.pallas import tpu_sc as plsc

B = 4096
N_CAT = 8
CAT_VOCAB = 1000
NC, NS = 2, 16
NW = NC * NS
BPW = B // NW
ID_W = 32
NUM_W = 64
CAT_W = 16
EMB_W = ID_W + N_CAT * CAT_W
HID = 128
OUT_W = 64
LANES = 16


@functools.lru_cache(maxsize=None)
def _make_sc_gather():
    mesh = plsc.VectorSubcoreMesh(core_axis_name="c", subcore_axis_name="s")

    @functools.partial(
        pl.kernel,
        mesh=mesh,
        out_type=(
            jax.ShapeDtypeStruct((NW, N_CAT * BPW, CAT_W), jnp.float32),
            jax.ShapeDtypeStruct((B, 128), jnp.float32),
        ),
        scratch_types=[
            pltpu.VMEM((BPW,), jnp.int32),
            pltpu.VMEM((N_CAT, BPW), jnp.int32),
            pltpu.VMEM((BPW, ID_W), jnp.float32),
            pltpu.VMEM((BPW, NUM_W), jnp.float32),
            pltpu.VMEM((N_CAT * BPW, CAT_W), jnp.float32),
            pltpu.SemaphoreType.DMA,
            pltpu.SemaphoreType.DMA,
        ],
        compiler_params=pltpu.CompilerParams(use_tc_tiling_on_sc=False),
    )
    def _sc(idt_hbm, cat_hbm, uid_hbm, cf_hbm, num_hbm, outc_hbm, outn_hbm,
            ids_v, cidx_v, id_rows, num_v, cat_rows, sem, sem2):
        wid = lax.axis_index("s") * NC + lax.axis_index("c")
        base = wid * BPW
        idx_loads = [
            pltpu.async_copy(uid_hbm.at[pl.ds(base, BPW)], ids_v, sem),
            pltpu.async_copy(cf_hbm.at[wid], cidx_v, sem),
        ]
        num_load = pltpu.async_copy(num_hbm.at[pl.ds(base, BPW)], num_v, sem2)
        for c in idx_loads:
            c.wait()
        offs = CAT_VOCAB * lax.rem(lax.iota(jnp.int32, LANES),
                                   jnp.full((LANES,), N_CAT, jnp.int32))
        for j in range(N_CAT):
            for k in range(BPW // LANES):
                sl = pl.ds(LANES * k, LANES)
                cidx_v[j, sl] = cidx_v[j, sl] + offs
        copies = [pltpu.async_copy(idt_hbm.at[ids_v], id_rows, sem)]
        for j in range(N_CAT):
            copies.append(
                pltpu.async_copy(cat_hbm.at[cidx_v.at[j]],
                                 cat_rows.at[pl.ds(BPW * j, BPW)], sem)
            )
        copies[0].wait()
        stores = [
            pltpu.async_copy(id_rows,
                             outn_hbm.at[pl.ds(base, BPW), pl.ds(0, ID_W)],
                             sem2)
        ]
        num_load.wait()
        stores.append(
            pltpu.async_copy(num_v,
                             outn_hbm.at[pl.ds(base, BPW), pl.ds(ID_W, NUM_W)],
                             sem2)
        )
        for j in range(N_CAT):
            copies[1 + j].wait()
        stores.append(pltpu.async_copy(cat_rows, outc_hbm.at[wid], sem2))
        for st in stores:
            st.wait()

    return _sc


def _mlp_body(c_ref, n_ref, w1_ref, b1_ref, w2_ref, b2_ref, o_ref):
    c = c_ref[...]
    n = n_ref[...]
    w1 = w1_ref[...]
    h = jnp.dot(n[:, 0:ID_W], w1[0:ID_W], preferred_element_type=jnp.float32)
    h = h + jnp.dot(c, w1[ID_W:ID_W + N_CAT * CAT_W],
                    preferred_element_type=jnp.float32)
    h = h + jnp.dot(n[:, ID_W:ID_W + NUM_W], w1[EMB_W:],
                    preferred_element_type=jnp.float32)
    h = jnp.maximum(h + b1_ref[...], 0.0)
    out = jnp.dot(h, w2_ref[...], preferred_element_type=jnp.float32) + b2_ref[...]
    ss = jnp.sum(out * out, axis=1, keepdims=True)
    o_ref[...] = out / jnp.maximum(jnp.sqrt(ss), 1e-12)


def _mlp(outc, outn, w1, b1, w2, b2, blk=2048):
    grid = (B // blk,)
    return pl.pallas_call(
        _mlp_body,
        grid=grid,
        in_specs=[
            pl.BlockSpec((blk, 128), lambda i: (i, 0)),
            pl.BlockSpec((blk, 128), lambda i: (i, 0)),
            pl.BlockSpec((EMB_W + NUM_W, HID), lambda i: (0, 0)),
            pl.BlockSpec((HID,), lambda i: (0,)),
            pl.BlockSpec((HID, OUT_W), lambda i: (0, 0)),
            pl.BlockSpec((OUT_W,), lambda i: (0,)),
        ],
        out_specs=pl.BlockSpec((blk, OUT_W), lambda i: (i, 0)),
        out_shape=jax.ShapeDtypeStruct((B, OUT_W), jnp.float32),
    )(outc, outn, w1, b1, w2, b2)


def kernel(user_ids, user_cat_feats, user_numeric_feats, user_id_table,
           cat_tables, W1, b1, W2, b2):
    uid = user_ids.astype(jnp.int32)
    cf = user_cat_feats.astype(jnp.int32).reshape(NW, N_CAT, BPW)
    catf = cat_tables.reshape(N_CAT * CAT_VOCAB, CAT_W)
    outc4, outn = _make_sc_gather()(
        user_id_table, catf, uid, cf, user_numeric_feats)
    return _mlp(outc4.reshape(B, 128), outn, W1, b1, W2, b2)


# speedup vs baseline: 1.0423x; 1.0423x over previous
import functools

import jax
import jax.numpy as jnp
from jax import lax
from jax.experimental import pallas as pl
from jax.experimental.pallas import tpu as pltpu
from jax.experimental.pallas import tpu_sc as plsc

B = 4096
N_CAT = 8
CAT_VOCAB = 1000
NC, NS = 2, 16
NW = NC * NS
BPW = B // NW
ID_W = 32
NUM_W = 64
CAT_W = 16
EMB_W = ID_W + N_CAT * CAT_W
HID = 128
OUT_W = 64
LANES = 16


@functools.lru_cache(maxsize=None)
def _make_sc_gather():
    mesh = plsc.VectorSubcoreMesh(core_axis_name="c", subcore_axis_name="s")

    @functools.partial(
        pl.kernel,
        mesh=mesh,
        out_type=(
            jax.ShapeDtypeStruct((NW, N_CAT * BPW, CAT_W), jnp.float32),
            jax.ShapeDtypeStruct((B, 128), jnp.float32),
        ),
        scratch_types=[
            pltpu.VMEM((BPW,), jnp.int32),
            pltpu.VMEM((N_CAT, BPW), jnp.int32),
            pltpu.VMEM((BPW, ID_W), jnp.float32),
            pltpu.VMEM((N_CAT * BPW, CAT_W), jnp.float32),
            pltpu.SemaphoreType.DMA,
            pltpu.SemaphoreType.DMA,
        ],
        compiler_params=pltpu.CompilerParams(use_tc_tiling_on_sc=False),
    )
    def _sc(idt_hbm, cat_hbm, uid_hbm, cf_hbm, outc_hbm, outn_hbm,
            ids_v, cidx_v, id_rows, cat_rows, sem, sem2):
        wid = lax.axis_index("s") * NC + lax.axis_index("c")
        base = wid * BPW
        idx_loads = [
            pltpu.async_copy(uid_hbm.at[pl.ds(base, BPW)], ids_v, sem),
            pltpu.async_copy(cf_hbm.at[wid], cidx_v, sem),
        ]
        for c in idx_loads:
            c.wait()
        offs = CAT_VOCAB * lax.rem(lax.iota(jnp.int32, LANES),
                                   jnp.full((LANES,), N_CAT, jnp.int32))
        for j in range(N_CAT):
            for k in range(BPW // LANES):
                sl = pl.ds(LANES * k, LANES)
                cidx_v[j, sl] = cidx_v[j, sl] + offs
        copies = [pltpu.async_copy(idt_hbm.at[ids_v], id_rows, sem)]
        for j in range(N_CAT):
            copies.append(
                pltpu.async_copy(cat_hbm.at[cidx_v.at[j]],
                                 cat_rows.at[pl.ds(BPW * j, BPW)], sem)
            )
        copies[0].wait()
        stores = [
            pltpu.async_copy(id_rows,
                             outn_hbm.at[pl.ds(base, BPW), pl.ds(0, ID_W)],
                             sem2)
        ]
        for j in range(N_CAT):
            copies[1 + j].wait()
            stores.append(
                pltpu.async_copy(
                    cat_rows.at[pl.ds(BPW * j, BPW)],
                    outc_hbm.at[wid].at[pl.ds(BPW * j, BPW)],
                    sem2,
                )
            )
        for st in stores:
            st.wait()

    return _sc


def _mlp_body(c_ref, n_ref, num_ref, w1_ref, b1_ref, w2_ref, b2_ref, o_ref):
    c = c_ref[...]
    n = n_ref[...]
    w1 = w1_ref[...]
    h = jnp.dot(n[:, 0:ID_W], w1[0:ID_W], preferred_element_type=jnp.float32)
    h = h + jnp.dot(c, w1[ID_W:ID_W + N_CAT * CAT_W],
                    preferred_element_type=jnp.float32)
    h = h + jnp.dot(num_ref[...], w1[EMB_W:],
                    preferred_element_type=jnp.float32)
    h = jnp.maximum(h + b1_ref[...], 0.0)
    out = jnp.dot(h, w2_ref[...], preferred_element_type=jnp.float32) + b2_ref[...]
    ss = jnp.sum(out * out, axis=1, keepdims=True)
    o_ref[...] = out / jnp.maximum(jnp.sqrt(ss), 1e-12)


def _mlp(outc, outn, numeric, w1, b1, w2, b2, blk=2048):
    grid = (B // blk,)
    return pl.pallas_call(
        _mlp_body,
        grid=grid,
        in_specs=[
            pl.BlockSpec((blk, 128), lambda i: (i, 0)),
            pl.BlockSpec((blk, 128), lambda i: (i, 0)),
            pl.BlockSpec((blk, NUM_W), lambda i: (i, 0)),
            pl.BlockSpec((EMB_W + NUM_W, HID), lambda i: (0, 0)),
            pl.BlockSpec((HID,), lambda i: (0,)),
            pl.BlockSpec((HID, OUT_W), lambda i: (0, 0)),
            pl.BlockSpec((OUT_W,), lambda i: (0,)),
        ],
        out_specs=pl.BlockSpec((blk, OUT_W), lambda i: (i, 0)),
        out_shape=jax.ShapeDtypeStruct((B, OUT_W), jnp.float32),
    )(outc, outn, numeric, w1, b1, w2, b2)


def kernel(user_ids, user_cat_feats, user_numeric_feats, user_id_table,
           cat_tables, W1, b1, W2, b2):
    uid = user_ids.astype(jnp.int32)
    cf = user_cat_feats.astype(jnp.int32).reshape(NW, N_CAT, BPW)
    catf = cat_tables.reshape(N_CAT * CAT_VOCAB, CAT_W)
    outc4, outn = _make_sc_gather()(user_id_table, catf, uid, cf)
    return _mlp(outc4.reshape(B, 128), outn, user_numeric_feats,
                W1, b1, W2, b2)


# speedup vs baseline: 1.0437x; 1.0014x over previous
import functools

import jax
import jax.numpy as jnp
from jax import lax
from jax.experimental import pallas as pl
from jax.experimental.pallas import tpu as pltpu
from jax.experimental.pallas import tpu_sc as plsc

B = 4096
N_CAT = 8
CAT_VOCAB = 1000
NC, NS = 2, 16
NW = NC * NS
BPW = B // NW
ID_W = 32
NUM_W = 64
CAT_W = 16
EMB_W = ID_W + N_CAT * CAT_W
HID = 128
OUT_W = 64
LANES = 16


@functools.lru_cache(maxsize=None)
def _make_sc_gather():
    mesh = plsc.VectorSubcoreMesh(core_axis_name="c", subcore_axis_name="s")

    @functools.partial(
        pl.kernel,
        mesh=mesh,
        out_type=(
            jax.ShapeDtypeStruct((NW, N_CAT * BPW, CAT_W), jnp.float32),
            jax.ShapeDtypeStruct((B, 128), jnp.float32),
        ),
        scratch_types=[
            pltpu.VMEM((BPW,), jnp.int32),
            pltpu.VMEM((N_CAT, BPW), jnp.int32),
            pltpu.VMEM((BPW, ID_W), jnp.float32),
            pltpu.VMEM((N_CAT * BPW, CAT_W), jnp.float32),
            pltpu.SemaphoreType.DMA,
            pltpu.SemaphoreType.DMA,
        ],
        compiler_params=pltpu.CompilerParams(use_tc_tiling_on_sc=False),
    )
    def _sc(idt_hbm, cat_hbm, uid_hbm, cf_hbm, outc_hbm, outn_hbm,
            ids_v, cidx_v, id_rows, cat_rows, sem, sem2):
        wid = lax.axis_index("s") * NC + lax.axis_index("c")
        base = wid * BPW
        idx_loads = [
            pltpu.async_copy(uid_hbm.at[pl.ds(base, BPW)], ids_v, sem),
            pltpu.async_copy(cf_hbm.at[wid], cidx_v, sem),
        ]
        for c in idx_loads:
            c.wait()
        copies = [pltpu.async_copy(idt_hbm.at[ids_v], id_rows, sem)]
        offs = CAT_VOCAB * lax.rem(lax.iota(jnp.int32, LANES),
                                   jnp.full((LANES,), N_CAT, jnp.int32))
        for j in range(N_CAT):
            for k in range(BPW // LANES):
                sl = pl.ds(LANES * k, LANES)
                cidx_v[j, sl] = cidx_v[j, sl] + offs
            copies.append(
                pltpu.async_copy(cat_hbm.at[cidx_v.at[j]],
                                 cat_rows.at[pl.ds(BPW * j, BPW)], sem)
            )
        copies[0].wait()
        stores = [
            pltpu.async_copy(id_rows,
                             outn_hbm.at[pl.ds(base, BPW), pl.ds(0, ID_W)],
                             sem2)
        ]
        for j in range(N_CAT):
            copies[1 + j].wait()
            stores.append(
                pltpu.async_copy(
                    cat_rows.at[pl.ds(BPW * j, BPW)],
                    outc_hbm.at[wid].at[pl.ds(BPW * j, BPW)],
                    sem2,
                )
            )
        for st in stores:
            st.wait()

    return _sc


def _mlp_body(c_ref, n_ref, num_ref, w1_ref, b1_ref, w2_ref, b2_ref, o_ref):
    c = c_ref[...]
    n = n_ref[...]
    w1 = w1_ref[...]
    h = jnp.dot(n[:, 0:ID_W], w1[0:ID_W], preferred_element_type=jnp.float32)
    h = h + jnp.dot(c, w1[ID_W:ID_W + N_CAT * CAT_W],
                    preferred_element_type=jnp.float32)
    h = h + jnp.dot(num_ref[...], w1[EMB_W:],
                    preferred_element_type=jnp.float32)
    h = jnp.maximum(h + b1_ref[...], 0.0)
    out = jnp.dot(h, w2_ref[...], preferred_element_type=jnp.float32) + b2_ref[...]
    ss = jnp.sum(out * out, axis=1, keepdims=True)
    o_ref[...] = out / jnp.maximum(jnp.sqrt(ss), 1e-12)


def _mlp(outc, outn, numeric, w1, b1, w2, b2, blk=2048):
    grid = (B // blk,)
    return pl.pallas_call(
        _mlp_body,
        grid=grid,
        in_specs=[
            pl.BlockSpec((blk, 128), lambda i: (i, 0)),
            pl.BlockSpec((blk, 128), lambda i: (i, 0)),
            pl.BlockSpec((blk, NUM_W), lambda i: (i, 0)),
            pl.BlockSpec((EMB_W + NUM_W, HID), lambda i: (0, 0)),
            pl.BlockSpec((HID,), lambda i: (0,)),
            pl.BlockSpec((HID, OUT_W), lambda i: (0, 0)),
            pl.BlockSpec((OUT_W,), lambda i: (0,)),
        ],
        out_specs=pl.BlockSpec((blk, OUT_W), lambda i: (i, 0)),
        out_shape=jax.ShapeDtypeStruct((B, OUT_W), jnp.float32),
    )(outc, outn, numeric, w1, b1, w2, b2)


def kernel(user_ids, user_cat_feats, user_numeric_feats, user_id_table,
           cat_tables, W1, b1, W2, b2):
    uid = user_ids.astype(jnp.int32)
    cf = user_cat_feats.astype(jnp.int32).reshape(NW, N_CAT, BPW)
    catf = cat_tables.reshape(N_CAT * CAT_VOCAB, CAT_W)
    outc4, outn = _make_sc_gather()(user_id_table, catf, uid, cf)
    return _mlp(outc4.reshape(B, 128), outn, user_numeric_feats,
                W1, b1, W2, b2)
